# baseline (speedup 1.0000x reference)
import functools

import jax
import jax.numpy as jnp
from jax import lax
from jax.experimental import pallas as pl
from jax.experimental.pallas import tpu as pltpu
from jax.experimental.pallas import tpu_sc as plsc

_CHUNK = 128


@functools.lru_cache(maxsize=None)
def _build(B, V, D, idx_dtype):
    info = plsc.get_sparse_core_info()
    nw = info.num_cores * info.num_subcores
    b_per_w = B // nw
    nchunk = b_per_w // _CHUNK
    mesh = plsc.VectorSubcoreMesh(core_axis_name="c", subcore_axis_name="s")

    @functools.partial(
        pl.kernel,
        mesh=mesh,
        compiler_params=pltpu.CompilerParams(use_tc_tiling_on_sc=False),
        out_type=jax.ShapeDtypeStruct((B, D), jnp.float32),
        scratch_types=[
            pltpu.VMEM((nchunk, _CHUNK), jnp.int32),
            pltpu.VMEM((b_per_w, D), jnp.float32),
            pltpu.SemaphoreType.DMA,
        ],
    )
    def gather_kernel(idx_hbm, table_hbm, out_hbm, idx_v, rows_v, sem):
        wid = lax.axis_index("s") * info.num_cores + lax.axis_index("c")
        base = wid * b_per_w
        pltpu.sync_copy(idx_hbm.at[wid], idx_v)
        copies = [
            pltpu.async_copy(
                table_hbm.at[idx_v.at[j]],
                rows_v.at[pl.ds(j * _CHUNK, _CHUNK)],
                sem,
            )
            for j in range(nchunk)
        ]
        for c in copies:
            c.wait()
        pltpu.sync_copy(rows_v, out_hbm.at[pl.ds(base, b_per_w)])

    def run(indexes, entity_table):
        idx3 = indexes.astype(jnp.int32).reshape(nw, nchunk, _CHUNK)
        return gather_kernel(idx3, entity_table)

    return run


def kernel(indexes, entity_table):
    (B,) = indexes.shape
    V, D = entity_table.shape
    return _build(B, V, D, indexes.dtype.name)(indexes, entity_table)
